# baseline (speedup 1.0000x reference)
import functools

import jax
import jax.numpy as jnp
from jax.experimental import pallas as pl
from jax.experimental.pallas import tpu as pltpu

_KT = 512
_LANE = 128


def _dot_k0(x, y):
    return jax.lax.dot_general(
        x, y, (((0,), (0,)), ((), ())), preferred_element_type=jnp.float32
    )


def _start_a_copies(a_hbm, at_scr, sems, block, slot, kt, sgrp):
    for i in range(sgrp):
        pltpu.make_async_copy(
            a_hbm.at[pl.ds(block * kt, kt), i, :],
            at_scr.at[slot, i],
            sems.at[slot, i],
        ).start()


def _wait_a_copies(at_scr, sems, slot, sgrp):
    for i in range(sgrp):
        pltpu.make_async_copy(
            at_scr.at[slot, i],
            at_scr.at[slot, i],
            sems.at[slot, i],
        ).wait()


def _body(
    a_hbm,
    p_ref,
    wt_ref,
    wpe_ref,
    b_ref,
    at_tail_ref,
    wt_tail_ref,
    o_ref,
    at_scr,
    sems,
    *,
    kt,
    sgrp,
):
    k = pl.program_id(0)
    nk = pl.num_programs(0)
    slot = jax.lax.rem(k, 2)

    @pl.when(k == 0)
    def _first_fetch():
        _start_a_copies(a_hbm, at_scr, sems, 0, 0, kt, sgrp)

    @pl.when(k + 1 < nk)
    def _next_fetch():
        _start_a_copies(a_hbm, at_scr, sems, k + 1, 1 - slot, kt, sgrp)

    @pl.when(k == 0)
    def _init():
        p = p_ref[...].astype(jnp.bfloat16)
        wp = wpe_ref[...].astype(jnp.bfloat16)
        acc = jax.lax.dot_general(
            p, wp, (((1,), (1,)), ((), ())), preferred_element_type=jnp.float32
        )
        o_ref[...] = acc + b_ref[...]
        wtl = wt_tail_ref[...].astype(jnp.bfloat16)
        for i in range(sgrp):
            o_ref[pl.ds(i * _LANE, _LANE), :] += _dot_k0(
                at_tail_ref[i].astype(jnp.bfloat16), wtl
            )

    _wait_a_copies(at_scr, sems, slot, sgrp)
    w = wt_ref[...].astype(jnp.bfloat16)
    for i in range(sgrp):
        a = at_scr[slot, i].astype(jnp.bfloat16)
        o_ref[pl.ds(i * _LANE, _LANE), :] += _dot_k0(a, w)


def kernel(input_ids, position_ids, W_wte, b_wte, W_wpe, b_wpe):
    b, s, v = input_ids.shape
    d = W_wte.shape[0]
    npos = position_ids.shape[-1]
    m = b * s
    sgrp = m // _LANE
    nk = v // _KT
    vmain = nk * _KT
    a3 = jnp.transpose(input_ids, (2, 0, 1)).reshape(v, sgrp, _LANE)
    wt = jnp.transpose(W_wte)
    p2 = position_ids.reshape(m, npos)
    bias = (b_wte + b_wpe).reshape(1, d)
    at_tail = jnp.transpose(a3[vmain:], (1, 0, 2))
    wt_tail = wt[vmain:]
    out = pl.pallas_call(
        functools.partial(_body, kt=_KT, sgrp=sgrp),
        grid=(nk,),
        in_specs=[
            pl.BlockSpec(memory_space=pltpu.MemorySpace.HBM),
            pl.BlockSpec((m, npos), lambda k: (0, 0)),
            pl.BlockSpec((_KT, d), lambda k: (k, 0)),
            pl.BlockSpec((d, npos), lambda k: (0, 0)),
            pl.BlockSpec((1, d), lambda k: (0, 0)),
            pl.BlockSpec((sgrp, v - vmain, _LANE), lambda k: (0, 0, 0)),
            pl.BlockSpec((v - vmain, d), lambda k: (0, 0)),
        ],
        out_specs=pl.BlockSpec((m, d), lambda k: (0, 0)),
        out_shape=jax.ShapeDtypeStruct((m, d), jnp.float32),
        scratch_shapes=[
            pltpu.VMEM((2, sgrp, _KT, _LANE), jnp.float32),
            pltpu.SemaphoreType.DMA((2, sgrp)),
        ],
        compiler_params=pltpu.CompilerParams(
            dimension_semantics=("arbitrary",)
        ),
    )(a3, p2, wt, W_wpe, bias, at_tail, wt_tail)
    return out.reshape(b, s, d)
